# Initial kernel scaffold; baseline (speedup 1.0000x reference)
import jax
import jax.numpy as jnp
from jax.experimental import pallas as pl


def kernel(x, mask_token):
    raise NotImplementedError("write your pallas kernel here")


# speedup vs baseline: 1.3316x; 1.3316x over previous
import functools

import numpy as np
import jax
import jax.numpy as jnp
from jax.experimental import pallas as pl


def _masked_copy_body(idx, blk, x_ref, mt_ref, o_ref):
    i = pl.program_id(0)
    rows = jax.lax.broadcasted_iota(jnp.int32, (blk, 1), 0) + i * blk
    m = (rows == idx[0]) | (rows == idx[1]) | (rows == idx[2])
    o_ref[...] = jnp.where(m[None, :, :], mt_ref[...], x_ref[...])


def kernel(x, mask_token):
    B, S, E = x.shape
    with jax.ensure_compile_time_eval():
        idx_arr = jax.random.choice(
            jax.random.key(42), S, shape=(3,), replace=False)
        idx = tuple(int(v) for v in np.asarray(idx_arr))

    blk = 256
    grid = (S // blk,)
    return pl.pallas_call(
        functools.partial(_masked_copy_body, idx, blk),
        grid=grid,
        in_specs=[
            pl.BlockSpec((B, blk, E), lambda i: (0, i, 0)),
            pl.BlockSpec((1, 1, E), lambda i: (0, 0, 0)),
        ],
        out_specs=pl.BlockSpec((B, blk, E), lambda i: (0, i, 0)),
        out_shape=jax.ShapeDtypeStruct((B, S, E), x.dtype),
    )(x, mask_token)


# speedup vs baseline: 1.3543x; 1.0170x over previous
import functools

import numpy as np
import jax
import jax.numpy as jnp
from jax.experimental import pallas as pl


def _masked_copy_body(idx, blk, x_ref, mt_ref, o_ref):
    i = pl.program_id(0)
    rows = jax.lax.broadcasted_iota(jnp.int32, (blk, 1), 0) + i * blk
    m = (rows == idx[0]) | (rows == idx[1]) | (rows == idx[2])
    o_ref[...] = jnp.where(m[None, :, :], mt_ref[...], x_ref[...])


def kernel(x, mask_token):
    B, S, E = x.shape
    with jax.ensure_compile_time_eval():
        idx_arr = jax.random.choice(
            jax.random.key(42), S, shape=(3,), replace=False)
        idx = tuple(int(v) for v in np.asarray(idx_arr))

    blk = 512
    grid = (S // blk,)
    return pl.pallas_call(
        functools.partial(_masked_copy_body, idx, blk),
        grid=grid,
        in_specs=[
            pl.BlockSpec((B, blk, E), lambda i: (0, i, 0)),
            pl.BlockSpec((1, 1, E), lambda i: (0, 0, 0)),
        ],
        out_specs=pl.BlockSpec((B, blk, E), lambda i: (0, i, 0)),
        out_shape=jax.ShapeDtypeStruct((B, S, E), x.dtype),
    )(x, mask_token)
